# baseline (speedup 1.0000x reference)
import functools

import jax
import jax.numpy as jnp
from jax import lax
from jax.experimental import pallas as pl
from jax.experimental.pallas import tpu as pltpu
from jax.experimental.pallas import tpu_sc as plsc

_NUM_BINS = 15
_LANES = 128
_K = 256
_NW = 32


def _lane_fold(a):
    t = a.shape[1]
    while t > _LANES:
        half = t // 2
        a = a[:, :half] + a[:, half:]
        t = half
    return a


def _tc_body(nb, c_dim, bnd_ref, sm_ref, lab_ref,
             ocnt_ref, oacc_ref, ocfs_ref, cnt_ref, acc_ref, cfs_ref):
    j = pl.program_id(0)

    @pl.when(j == 0)
    def _init():
        zeros = jnp.zeros(cnt_ref.shape, jnp.float32)
        cnt_ref[...] = zeros
        acc_ref[...] = zeros
        cfs_ref[...] = zeros

    best = sm_ref[0]
    sbest = jnp.full(best.shape, float(c_dim - 1), jnp.float32)
    for c in range(1, c_dim):
        xc = sm_ref[c]
        gt = xc > best
        best = jnp.maximum(xc, best)
        sbest = jnp.where(gt, float(c_dim - 1 - c), sbest)

    target = float(c_dim - 1) - lab_ref[...].astype(jnp.float32)
    corr = (sbest == target).astype(jnp.float32)

    for i in range(_NUM_BINS + 1):
        m = best > bnd_ref[i]
        cnt_ref[i] += _lane_fold(m.astype(jnp.float32))
        acc_ref[i] += _lane_fold(jnp.where(m, corr, 0.0))
        cfs_ref[i] += _lane_fold(jnp.where(m, best, 0.0))

    @pl.when(j == nb - 1)
    def _fin():
        ocnt_ref[...] = jnp.sum(cnt_ref[...], axis=2)
        oacc_ref[...] = jnp.sum(acc_ref[...], axis=2)
        ocfs_ref[...] = jnp.sum(cfs_ref[...], axis=2)


def _sc_body(n0, w_cols, c_dim, sm_hbm, lab_hbm, bb_hbm, out_hbm,
             xbuf_a, lbuf_a, xbuf_b, lbuf_b, bbuf, cnt_h, acc_h, cfs_h,
             sem_a, sem_b):
    wid = lax.axis_index("s") * 2 + lax.axis_index("c")
    base = n0 + wid * w_cols
    pltpu.sync_copy(bb_hbm, bbuf)
    z16 = jnp.zeros((16,), jnp.float32)
    for i in range(8):
        cnt_h[pl.ds(i * 16, 16)] = z16
        acc_h[pl.ds(i * 16, 16)] = z16
        cfs_h[pl.ds(i * 16, 16)] = z16

    def start(s, xbuf, lbuf, sem):
        off = base + s * _K
        pltpu.async_copy(sm_hbm.at[:, :, pl.ds(off, _K)], xbuf, sem)
        pltpu.async_copy(lab_hbm.at[:, pl.ds(off, _K)], lbuf, sem)

    def wait(s, xbuf, lbuf, sem):
        off = base + s * _K
        pltpu.make_async_copy(sm_hbm.at[:, :, pl.ds(off, _K)], xbuf, sem).wait()
        pltpu.make_async_copy(lab_hbm.at[:, pl.ds(off, _K)], lbuf, sem).wait()

    def compute(xbuf, lbuf):
        def inner(g, c2):
            sl = pl.ds(g * 16, 16)
            for b in range(8):
                best = xbuf[0, b, sl]
                sbest = jnp.full((16,), float(c_dim - 1), jnp.float32)
                for c in range(1, c_dim):
                    xc = xbuf[c, b, sl]
                    gt = xc > best
                    best = jnp.maximum(xc, best)
                    sbest = jnp.where(gt, float(c_dim - 1 - c), sbest)
                labf = lbuf[b, sl].astype(jnp.float32)
                corr = jnp.where(sbest == (float(c_dim - 1) - labf),
                                 1.0, 0.0).astype(jnp.float32)
                kf = jnp.zeros((16,), jnp.float32)
                for i in range(_NUM_BINS):
                    kf += jnp.where(best > bbuf[i], 1.0, 0.0)
                valid = kf > 0.0
                ki = kf.astype(jnp.int32) - 1
                idx = jnp.where(valid, ki, _NUM_BINS) * 8 + b
                ones = jnp.ones((16,), jnp.float32)
                plsc.addupdate_scatter(cnt_h, [idx], ones)
                plsc.addupdate_scatter(acc_h, [idx], corr)
                plsc.addupdate_scatter(cfs_h, [idx], best)
            return c2

        lax.fori_loop(0, _K // 16, inner, 0)

    nsteps = w_cols // _K

    start(0, xbuf_a, lbuf_a, sem_a)

    def pair(k, carry):
        s0 = 2 * k
        start(s0 + 1, xbuf_b, lbuf_b, sem_b)
        wait(s0, xbuf_a, lbuf_a, sem_a)
        compute(xbuf_a, lbuf_a)
        nxt = lax.rem(s0 + 2, nsteps)
        start(nxt, xbuf_a, lbuf_a, sem_a)
        wait(s0 + 1, xbuf_b, lbuf_b, sem_b)
        compute(xbuf_b, lbuf_b)
        return carry

    lax.fori_loop(0, nsteps // 2, pair, 0)
    wait(0, xbuf_a, lbuf_a, sem_a)

    pltpu.sync_copy(cnt_h, out_hbm.at[pl.ds((wid * 3) * _LANES, _LANES)])
    pltpu.sync_copy(acc_h, out_hbm.at[pl.ds((wid * 3 + 1) * _LANES, _LANES)])
    pltpu.sync_copy(cfs_h, out_hbm.at[pl.ds((wid * 3 + 2) * _LANES, _LANES)])


def _comb_body(n_total, cnt_ref, acc_ref, cfs_ref, scp_ref, out_ref):
    scp = jnp.sum(scp_ref[...], axis=0)
    scnt = scp[0][:_NUM_BINS]
    sacc = scp[1][:_NUM_BINS]
    scfs = scp[2][:_NUM_BINS]
    cnt = cnt_ref[...]
    accs = acc_ref[...]
    cfss = cfs_ref[...]
    count = cnt[:-1] - cnt[1:] + scnt
    asum = accs[:-1] - accs[1:] + sacc
    csum = cfss[:-1] - cfss[1:] + scfs
    prop = count / float(n_total)
    denom = jnp.maximum(count, 1.0)
    contrib = jnp.where(count > 0.0,
                        jnp.abs(csum / denom - asum / denom) * prop, 0.0)
    ece = jnp.sum(contrib, axis=0)
    out_ref[...] = jnp.broadcast_to(ece[:, None], out_ref.shape)


def kernel(edl_u, softmax, label):
    del edl_u
    b_dim, c_dim, n = softmax.shape
    sm_t = jnp.transpose(softmax, (1, 0, 2))
    label = label.astype(jnp.int32)
    bnd = jnp.linspace(0.0, 1.0, _NUM_BINS + 1, dtype=jnp.float32)

    tile = 8192
    while n % tile:
        tile //= 2

    n_sc = n // 16
    w_cols = n_sc // _NW
    if (w_cols == 0 or w_cols % (2 * _K) or n_sc % _NW
            or (n - n_sc) % tile):
        n_sc, w_cols = 0, 0
    n_tc = n - n_sc
    nb = n_tc // tile

    body = functools.partial(_tc_body, nb, c_dim)
    sums2 = jax.ShapeDtypeStruct((_NUM_BINS + 1, b_dim), jnp.float32)
    tc_cnt, tc_acc, tc_cfs = pl.pallas_call(
        body,
        grid=(nb,),
        in_specs=[
            pl.BlockSpec(memory_space=pltpu.SMEM),
            pl.BlockSpec((c_dim, b_dim, tile), lambda j: (0, 0, j)),
            pl.BlockSpec((b_dim, tile), lambda j: (0, j)),
        ],
        out_specs=[pl.BlockSpec((_NUM_BINS + 1, b_dim), lambda j: (0, 0))
                   for _ in range(3)],
        out_shape=[sums2, sums2, sums2],
        scratch_shapes=[pltpu.VMEM((_NUM_BINS + 1, b_dim, _LANES),
                                   jnp.float32) for _ in range(3)],
    )(bnd, sm_t, label)

    if n_sc:
        bb_bcast = jnp.broadcast_to(bnd[:_NUM_BINS, None], (_NUM_BINS, 16))
        mesh = plsc.VectorSubcoreMesh(core_axis_name="c", subcore_axis_name="s")
        sc_fn = pl.kernel(
            functools.partial(_sc_body, n_tc, w_cols, c_dim),
            out_type=jax.ShapeDtypeStruct((_NW * 3 * _LANES,), jnp.float32),
            mesh=mesh,
            compiler_params=pltpu.CompilerParams(needs_layout_passes=False),
            scratch_types=[
                pltpu.VMEM((c_dim, b_dim, _K), jnp.float32),
                pltpu.VMEM((b_dim, _K), jnp.int32),
                pltpu.VMEM((c_dim, b_dim, _K), jnp.float32),
                pltpu.VMEM((b_dim, _K), jnp.int32),
                pltpu.VMEM((_NUM_BINS, 16), jnp.float32),
                pltpu.VMEM((_LANES,), jnp.float32),
                pltpu.VMEM((_LANES,), jnp.float32),
                pltpu.VMEM((_LANES,), jnp.float32),
                pltpu.SemaphoreType.DMA,
                pltpu.SemaphoreType.DMA,
            ],
        )
        sc_part = sc_fn(sm_t, label, bb_bcast).reshape(
            _NW, 3, _NUM_BINS + 1, b_dim)
    else:
        sc_part = jnp.zeros((_NW, 3, _NUM_BINS + 1, b_dim), jnp.float32)

    out = pl.pallas_call(
        functools.partial(_comb_body, n),
        out_shape=jax.ShapeDtypeStruct((b_dim, _LANES), jnp.float32),
    )(tc_cnt, tc_acc, tc_cfs, sc_part)
    return out[:, 0]


# speedup vs baseline: 1.2077x; 1.2077x over previous
import functools

import jax
import jax.numpy as jnp
from jax.experimental import pallas as pl
from jax.experimental.pallas import tpu as pltpu

_NUM_BINS = 15
_LANES = 128


def _lane_fold(a):
    t = a.shape[1]
    while t > _LANES:
        half = t // 2
        a = a[:, :half] + a[:, half:]
        t = half
    return a


def _ece_body(nb, c_dim, n_total, bnd_ref, sm_ref, lab_ref, out_ref,
              cnt_ref, acc_ref, cfs_ref):
    j = pl.program_id(0)

    @pl.when(j == 0)
    def _init():
        zeros = jnp.zeros(cnt_ref.shape, jnp.float32)
        cnt_ref[...] = zeros
        acc_ref[...] = zeros
        cfs_ref[...] = zeros

    best = sm_ref[0]
    sbest = jnp.full(best.shape, float(c_dim - 1), jnp.float32)
    for c in range(1, c_dim):
        xc = sm_ref[c]
        gt = xc > best
        best = jnp.maximum(xc, best)
        sbest = jnp.where(gt, float(c_dim - 1 - c), sbest)

    target = float(c_dim - 1) - lab_ref[...].astype(jnp.float32)
    corr = (sbest == target).astype(jnp.float32)

    for i in range(_NUM_BINS + 1):
        m = best > bnd_ref[i]
        cnt_ref[i] += _lane_fold(m.astype(jnp.float32))
        acc_ref[i] += _lane_fold(jnp.where(m, corr, 0.0))
        cfs_ref[i] += _lane_fold(jnp.where(m, best, 0.0))

    @pl.when(j == nb - 1)
    def _fin():
        cnt = jnp.sum(cnt_ref[...], axis=2)
        accs = jnp.sum(acc_ref[...], axis=2)
        cfss = jnp.sum(cfs_ref[...], axis=2)
        count = cnt[:-1] - cnt[1:]
        prop = count / float(n_total)
        denom = jnp.maximum(count, 1.0)
        acc_b = (accs[:-1] - accs[1:]) / denom
        cfs_b = (cfss[:-1] - cfss[1:]) / denom
        contrib = jnp.where(count > 0.0,
                            jnp.abs(cfs_b - acc_b) * prop, 0.0)
        ece = jnp.sum(contrib, axis=0)
        out_ref[...] = jnp.broadcast_to(ece[:, None], out_ref.shape)


def kernel(edl_u, softmax, label):
    del edl_u
    b_dim, c_dim, n = softmax.shape
    sm_t = jnp.transpose(softmax, (1, 0, 2))
    tile = 8192
    while n % tile:
        tile //= 2
    nb = n // tile

    label = label.astype(jnp.int32)
    bnd = jnp.linspace(0.0, 1.0, _NUM_BINS + 1, dtype=jnp.float32)

    body = functools.partial(_ece_body, nb, c_dim, n)
    out = pl.pallas_call(
        body,
        grid=(nb,),
        in_specs=[
            pl.BlockSpec(memory_space=pltpu.SMEM),
            pl.BlockSpec((c_dim, b_dim, tile), lambda j: (0, 0, j)),
            pl.BlockSpec((b_dim, tile), lambda j: (0, j)),
        ],
        out_specs=pl.BlockSpec((b_dim, _LANES), lambda j: (0, 0)),
        out_shape=jax.ShapeDtypeStruct((b_dim, _LANES), jnp.float32),
        scratch_shapes=[pltpu.VMEM((_NUM_BINS + 1, b_dim, _LANES),
                                   jnp.float32) for _ in range(3)],
    )(bnd, sm_t, label)
    return out[:, 0]


# speedup vs baseline: 1.3979x; 1.1575x over previous
import functools

import jax
import jax.numpy as jnp
from jax.experimental import pallas as pl
from jax.experimental.pallas import tpu as pltpu

_NUM_BINS = 15
_LANES = 128


def _lane_fold(a):
    t = a.shape[1]
    while t > _LANES:
        half = t // 2
        a = a[:, :half] + a[:, half:]
        t = half
    return a


def _ece_body(nb, c_dim, n_total, bnd_ref, sm_ref, lab_ref, out_ref,
              cnt_ref, acc_ref, cfs_ref):
    j = pl.program_id(0)

    @pl.when(j == 0)
    def _init():
        zeros = jnp.zeros(cnt_ref.shape, jnp.float32)
        cnt_ref[...] = zeros
        acc_ref[...] = zeros
        cfs_ref[...] = zeros

    best = sm_ref[0]
    sbest = jnp.full(best.shape, float(c_dim - 1), jnp.float32)
    for c in range(1, c_dim):
        xc = sm_ref[c]
        gt = xc > best
        best = jnp.maximum(xc, best)
        sbest = jnp.where(gt, float(c_dim - 1 - c), sbest)

    target = float(c_dim - 1) - lab_ref[...].astype(jnp.float32)
    corr = (sbest == target).astype(jnp.float32)

    t = best.shape[1]
    chunk = 2048 if t % 2048 == 0 else t
    for q in range(t // chunk):
        bq = best[:, q * chunk:(q + 1) * chunk]
        cq = corr[:, q * chunk:(q + 1) * chunk]
        for i in range(_NUM_BINS + 1):
            m = bq > bnd_ref[i]
            cnt_ref[i] += _lane_fold(m.astype(jnp.float32))
            acc_ref[i] += _lane_fold(jnp.where(m, cq, 0.0))
            cfs_ref[i] += _lane_fold(jnp.where(m, bq, 0.0))

    @pl.when(j == nb - 1)
    def _fin():
        cnt = jnp.sum(cnt_ref[...], axis=2)
        accs = jnp.sum(acc_ref[...], axis=2)
        cfss = jnp.sum(cfs_ref[...], axis=2)
        count = cnt[:-1] - cnt[1:]
        prop = count / float(n_total)
        denom = jnp.maximum(count, 1.0)
        acc_b = (accs[:-1] - accs[1:]) / denom
        cfs_b = (cfss[:-1] - cfss[1:]) / denom
        contrib = jnp.where(count > 0.0,
                            jnp.abs(cfs_b - acc_b) * prop, 0.0)
        ece = jnp.sum(contrib, axis=0)
        out_ref[...] = jnp.broadcast_to(ece[:, None], out_ref.shape)


def kernel(edl_u, softmax, label):
    del edl_u
    b_dim, c_dim, n = softmax.shape
    sm_t = jnp.transpose(softmax, (1, 0, 2))
    tile = 8192
    while n % tile:
        tile //= 2
    nb = n // tile

    label = label.astype(jnp.int32)
    bnd = jnp.linspace(0.0, 1.0, _NUM_BINS + 1, dtype=jnp.float32)

    body = functools.partial(_ece_body, nb, c_dim, n)
    out = pl.pallas_call(
        body,
        grid=(nb,),
        in_specs=[
            pl.BlockSpec(memory_space=pltpu.SMEM),
            pl.BlockSpec((c_dim, b_dim, tile), lambda j: (0, 0, j)),
            pl.BlockSpec((b_dim, tile), lambda j: (0, j)),
        ],
        out_specs=pl.BlockSpec((b_dim, _LANES), lambda j: (0, 0)),
        out_shape=jax.ShapeDtypeStruct((b_dim, _LANES), jnp.float32),
        scratch_shapes=[pltpu.VMEM((_NUM_BINS + 1, b_dim, _LANES),
                                   jnp.float32) for _ in range(3)],
    )(bnd, sm_t, label)
    return out[:, 0]
